# baseline (speedup 1.0000x reference)
import functools
import math

import jax
import jax.numpy as jnp
from jax import lax
from jax.experimental import pallas as pl
from jax.experimental.pallas import tpu as pltpu
from jax.experimental.pallas import tpu_sc as plsc

NC = 2
NS = 16
L = 16
NW = NC * NS

CHUNK = 1024
GSUB = CHUNK // 128


def _rsqrt_newton(x):
    i = plsc.bitcast(x, jnp.int32)
    i = jnp.int32(0x5F3759DF) - lax.shift_right_arithmetic(i, jnp.int32(1))
    y = plsc.bitcast(i, jnp.float32)
    for _ in range(3):
        y = y * (1.5 - 0.5 * x * y * y)
    return y


def _make_sc_kernel(n_idx, n_bags, d_model):
    assert d_model % L == 0 and n_bags % NW == 0
    bags_w = n_bags // NW
    dq = d_model // L
    stag_rows = bags_w + L
    mesh = plsc.VectorSubcoreMesh(core_axis_name="c", subcore_axis_name="s")
    out_scale = math.sqrt(d_model)

    @functools.partial(
        pl.kernel,
        mesh=mesh,
        compiler_params=pltpu.CompilerParams(
            needs_layout_passes=False, use_tc_tiling_on_sc=False),
        out_type=jax.ShapeDtypeStruct((n_bags * d_model,), jnp.float32),
        scratch_types=[
            pltpu.VMEM((CHUNK,), jnp.int32),
            pltpu.VMEM((CHUNK, d_model), jnp.float32),
            pltpu.VMEM((stag_rows * d_model,), jnp.float32),
            pltpu.VMEM((stag_rows,), jnp.float32),
            pltpu.VMEM((bags_w,), jnp.int32),
            pltpu.VMEM((48,), jnp.int32),
            pltpu.VMEM((CHUNK + L,), jnp.int32),
            pltpu.SemaphoreType.DMA,
        ],
    )
    def sc_kernel(ind_hbm, offs_hbm, bounds_hbm, weight_hbm, out_hbm,
                  idx_v, rows_v, staging, cnts, offs_v, bounds_v, delta,
                  sem):
        wid = lax.axis_index("s") * NC + lax.axis_index("c")
        bag_lo = pl.multiple_of(wid * bags_w, 8)

        zf = jnp.zeros((L,), jnp.float32)
        iota = lax.iota(jnp.int32, L)

        pltpu.sync_copy(offs_hbm.at[pl.ds(bag_lo, bags_w)], offs_v)
        pltpu.sync_copy(bounds_hbm, bounds_v)

        bv = bounds_v[pl.ds(wid, L)]
        row_start = bv[0]
        row_end = bv[1]
        base = lax.bitwise_and(row_start, jnp.int32(-8))
        nchunks = (row_end - base + (CHUNK - 1)) // CHUNK

        def _z(i, _):
            staging[pl.ds(i * L, L)] = zf
            return 0
        lax.fori_loop(0, stag_rows * d_model // L, _z, 0)

        def _zc(i, _):
            cnts[pl.ds(i * L, L)] = zf
            return 0
        lax.fori_loop(0, stag_rows // L, _zc, 0)

        def chunk_body(g, bagcum):
            r0 = pl.multiple_of(base + g * CHUNK, 8)

            pltpu.sync_copy(ind_hbm.at[pl.ds(r0, CHUNK)], idx_v)
            copies = []
            for k in range(GSUB):
                copies.append(pltpu.async_copy(
                    weight_hbm.at[idx_v.at[pl.ds(k * 128, 128)]],
                    rows_v.at[pl.ds(k * 128, 128)], sem))
            for c in copies:
                c.wait()

            def _zd(i, _):
                delta[pl.ds(i * L, L)] = jnp.zeros((L,), jnp.int32)
                return 0
            lax.fori_loop(0, CHUNK // L, _zd, 0)

            for m in range(bags_w // L):
                o = offs_v[pl.ds(m * L, L)]
                inr = jnp.logical_and(o >= r0, o < r0 + CHUNK)
                tgt = jnp.where(inr, o - r0, 0)
                vals = jnp.full((L,), m * L + 1, jnp.int32) + iota
                plsc.store_scatter(delta, [tgt], vals, mask=inr)

            def grp_body(j, gbag):
                rbase = j * L
                d = delta[pl.ds(rbase, L)]
                s = jnp.maximum(plsc.cummax(d), jnp.full((L,), gbag))
                gbag = s[L - 1]
                slot = jnp.where(s == jnp.int32(0), jnp.int32(bags_w), s - 1)

                rowids = rbase + iota
                parts = [zf for _ in range(8)]
                for c in range(d_model):
                    col = plsc.load_gather(
                        rows_v, [rowids, jnp.full((L,), c, jnp.int32)])
                    parts[c % 8] = parts[c % 8] + col * col
                n2 = ((parts[0] + parts[1]) + (parts[2] + parts[3])) + (
                    (parts[4] + parts[5]) + (parts[6] + parts[7])) + 1e-12
                scale = jnp.minimum(1.0, _rsqrt_newton(n2))

                iv = idx_v[pl.ds(rbase, L)]
                maskf = jnp.where(iv != jnp.int32(0), 1.0, 0.0)
                validf = jnp.where(r0 + rowids < row_end, 1.0, 0.0)
                cntc = maskf * validf
                coeff = scale * cntc

                sbase = slot * jnp.int32(d_model)
                for c in range(d_model):
                    col = plsc.load_gather(
                        rows_v, [rowids, jnp.full((L,), c, jnp.int32)])
                    plsc.addupdate_scatter(
                        staging, [sbase + jnp.int32(c)], col * coeff)
                plsc.addupdate_scatter(cnts, [slot], cntc)
                return gbag

            return lax.fori_loop(0, CHUNK // L, grp_body, bagcum)

        lax.fori_loop(0, nchunks, chunk_body, jnp.int32(0))

        def fin_body(jb, _):
            bb = jb * L + iota
            cvec = cnts[pl.ds(jb * L, L)]
            ivv = jnp.full((L,), out_scale, jnp.float32) / jnp.maximum(
                cvec, 1.0)
            pbase = bb * jnp.int32(d_model)
            for c in range(d_model):
                pos = pbase + jnp.int32(c)
                v = plsc.load_gather(staging, [pos]) * ivv
                plsc.store_scatter(staging, [pos], v)
            return 0
        lax.fori_loop(0, bags_w // L, fin_body, 0)

        obase = pl.multiple_of(bag_lo * d_model, 8)
        pltpu.sync_copy(staging.at[pl.ds(0, bags_w * d_model)],
                        out_hbm.at[pl.ds(obase, bags_w * d_model)])

    return sc_kernel


def kernel(indices, offsets, weight):
    n_idx = indices.shape[0]
    n_bags = offsets.shape[0]
    d_model = weight.shape[1]
    bags_w = n_bags // NW

    ind32 = indices.astype(jnp.int32)
    offs32 = offsets.astype(jnp.int32)
    is_last = jnp.concatenate(
        [offs32[1:] != offs32[:-1], jnp.ones((1,), bool)])
    spos = jnp.where(is_last, offs32, -1)
    bounds = jnp.concatenate(
        [offs32[::bags_w], jnp.full((48 - NW,), n_idx, jnp.int32)])
    ind_p = jnp.concatenate([ind32, jnp.zeros((CHUNK,), jnp.int32)])

    sc = _make_sc_kernel(n_idx, n_bags, d_model)
    return sc(ind_p, spos, bounds, weight).reshape(n_bags, d_model)


# speedup vs baseline: 2.5582x; 2.5582x over previous
import functools
import math

import jax
import jax.numpy as jnp
from jax import lax
from jax.experimental import pallas as pl
from jax.experimental.pallas import tpu as pltpu
from jax.experimental.pallas import tpu_sc as plsc

NC = 2
NS = 16
L = 16
NW = NC * NS

CHUNK = 1024
GSUB = CHUNK // 128


def _rsqrt_newton(x):
    i = plsc.bitcast(x, jnp.int32)
    i = jnp.int32(0x5F3759DF) - lax.shift_right_arithmetic(i, jnp.int32(1))
    y = plsc.bitcast(i, jnp.float32)
    for _ in range(2):
        y = y * (1.5 - 0.5 * x * y * y)
    return y


def _make_sc_kernel(n_idx, n_bags, d_model):
    assert d_model % L == 0 and n_bags % NW == 0
    bags_w = n_bags // NW
    dq = d_model // L
    stag_rows = bags_w + L
    mesh = plsc.VectorSubcoreMesh(core_axis_name="c", subcore_axis_name="s")
    out_scale = math.sqrt(d_model)

    @functools.partial(
        pl.kernel,
        mesh=mesh,
        compiler_params=pltpu.CompilerParams(
            needs_layout_passes=False, use_tc_tiling_on_sc=False),
        out_type=jax.ShapeDtypeStruct((n_bags, d_model), jnp.float32),
        scratch_types=[
            pltpu.VMEM((CHUNK,), jnp.int32),
            pltpu.VMEM((CHUNK, d_model), jnp.float32),
            pltpu.VMEM((stag_rows, d_model), jnp.float32),
            pltpu.VMEM((stag_rows, L), jnp.float32),
            pltpu.VMEM((bags_w,), jnp.int32),
            pltpu.VMEM((48,), jnp.int32),
            pltpu.VMEM((CHUNK + L,), jnp.int32),
            pltpu.SemaphoreType.DMA,
        ],
    )
    def sc_kernel(ind_hbm, offs_hbm, bounds_hbm, weight_hbm, out_hbm,
                  idx_v, rows_v, staging, cnts, offs_v, bounds_v, delta,
                  sem):
        wid = lax.axis_index("s") * NC + lax.axis_index("c")
        bag_lo = pl.multiple_of(wid * bags_w, 8)

        zf = jnp.zeros((L,), jnp.float32)
        iota = lax.iota(jnp.int32, L)

        pltpu.sync_copy(offs_hbm.at[pl.ds(bag_lo, bags_w)], offs_v)
        pltpu.sync_copy(bounds_hbm, bounds_v)

        bv = bounds_v[pl.ds(wid, L)]
        row_start = bv[0]
        row_end = bv[1]
        base = lax.bitwise_and(row_start, jnp.int32(-8))
        nchunks = (row_end - base + (CHUNK - 1)) // CHUNK

        def _z(i, _):
            for q in range(dq):
                staging[i, pl.ds(q * L, L)] = zf
            cnts[i, pl.ds(0, L)] = zf
            return 0
        lax.fori_loop(0, stag_rows, _z, 0)

        def chunk_body(g, bagcum):
            r0 = pl.multiple_of(base + g * CHUNK, 8)

            pltpu.sync_copy(ind_hbm.at[pl.ds(r0, CHUNK)], idx_v)
            copies = []
            for k in range(GSUB):
                copies.append(pltpu.async_copy(
                    weight_hbm.at[idx_v.at[pl.ds(k * 128, 128)]],
                    rows_v.at[pl.ds(k * 128, 128)], sem))
            for c in copies:
                c.wait()

            def _zd(i, _):
                delta[pl.ds(i * L, L)] = jnp.zeros((L,), jnp.int32)
                return 0
            lax.fori_loop(0, CHUNK // L, _zd, 0)

            for m in range(bags_w // L):
                o = offs_v[pl.ds(m * L, L)]
                inr = jnp.logical_and(o >= r0, o < r0 + CHUNK)
                tgt = jnp.where(inr, o - r0, 0)
                vals = jnp.full((L,), m * L + 1, jnp.int32) + iota
                plsc.store_scatter(delta, [tgt], vals, mask=inr)

            def grp_body(j, gbag):
                rbase = j * L
                d = delta[pl.ds(rbase, L)]
                s = jnp.maximum(plsc.cummax(d), jnp.full((L,), gbag))
                gbag = s[L - 1]
                slot = jnp.where(s == jnp.int32(0), jnp.int32(bags_w), s - 1)
                iv = idx_v[pl.ds(rbase, L)]

                for l in range(L):
                    r = rbase + l
                    vs = [rows_v[r, pl.ds(q * L, L)] for q in range(dq)]
                    sq = [v * v for v in vs]
                    while len(sq) > 1:
                        sq = [sq[i] + sq[i + 1] for i in range(0, len(sq), 2)]
                    n2s = jnp.sum(sq[0]) + 1e-12
                    mv = jnp.where(iv[l] != jnp.int32(0), 1.0, 0.0)
                    mv = mv * jnp.where(r0 + r < row_end, 1.0, 0.0)
                    cv = jnp.minimum(
                        1.0, _rsqrt_newton(jnp.full((L,), n2s))) * mv
                    sl = slot[l]
                    for q in range(dq):
                        plsc.addupdate(staging.at[sl, pl.ds(q * L, L)],
                                       cv * vs[q])
                    plsc.addupdate(cnts.at[sl, pl.ds(0, L)],
                                   jnp.full((L,), mv))
                return gbag

            return lax.fori_loop(0, CHUNK // L, grp_body, bagcum)

        lax.fori_loop(0, nchunks, chunk_body, jnp.int32(0))

        def fin_body(b, _):
            cvec = cnts[b, pl.ds(0, L)]
            ivv = jnp.full((L,), out_scale, jnp.float32) / jnp.maximum(
                cvec, 1.0)
            for q in range(dq):
                staging[b, pl.ds(q * L, L)] = (
                    staging[b, pl.ds(q * L, L)] * ivv)
            return 0
        lax.fori_loop(0, bags_w, fin_body, 0)

        pltpu.sync_copy(staging.at[pl.ds(0, bags_w)],
                        out_hbm.at[pl.ds(bag_lo, bags_w)])

    return sc_kernel


def kernel(indices, offsets, weight):
    n_idx = indices.shape[0]
    n_bags = offsets.shape[0]
    d_model = weight.shape[1]
    bags_w = n_bags // NW

    ind32 = indices.astype(jnp.int32)
    offs32 = offsets.astype(jnp.int32)
    is_last = jnp.concatenate(
        [offs32[1:] != offs32[:-1], jnp.ones((1,), bool)])
    spos = jnp.where(is_last, offs32, -1)
    bounds = jnp.concatenate(
        [offs32[::bags_w], jnp.full((48 - NW,), n_idx, jnp.int32)])
    ind_p = jnp.concatenate([ind32, jnp.zeros((CHUNK,), jnp.int32)])

    sc = _make_sc_kernel(n_idx, n_bags, d_model)
    return sc(ind_p, spos, bounds, weight)


# speedup vs baseline: 2.6422x; 1.0328x over previous
import functools
import math

import jax
import jax.numpy as jnp
from jax import lax
from jax.experimental import pallas as pl
from jax.experimental.pallas import tpu as pltpu
from jax.experimental.pallas import tpu_sc as plsc

NC = 2
NS = 16
L = 16
NW = NC * NS

CHUNK = 512
GSUB = CHUNK // 128


def _rsqrt_newton(x):
    i = plsc.bitcast(x, jnp.int32)
    i = jnp.int32(0x5F3759DF) - lax.shift_right_arithmetic(i, jnp.int32(1))
    y = plsc.bitcast(i, jnp.float32)
    for _ in range(2):
        y = y * (1.5 - 0.5 * x * y * y)
    return y


def _make_sc_kernel(n_idx, n_bags, d_model):
    assert d_model % L == 0 and n_bags % NW == 0
    bags_w = n_bags // NW
    dq = d_model // L
    stag_rows = bags_w + L
    mesh = plsc.VectorSubcoreMesh(core_axis_name="c", subcore_axis_name="s")
    out_scale = math.sqrt(d_model)

    @functools.partial(
        pl.kernel,
        mesh=mesh,
        compiler_params=pltpu.CompilerParams(
            needs_layout_passes=False, use_tc_tiling_on_sc=False),
        out_type=jax.ShapeDtypeStruct((n_bags, d_model), jnp.float32),
        scratch_types=[
            [pltpu.VMEM((CHUNK,), jnp.int32) for _ in range(2)],
            [pltpu.VMEM((CHUNK, d_model), jnp.float32)
             for _ in range(2)],
            pltpu.VMEM((stag_rows, d_model), jnp.float32),
            pltpu.VMEM((stag_rows, L), jnp.float32),
            pltpu.VMEM((bags_w,), jnp.int32),
            pltpu.VMEM((48,), jnp.int32),
            pltpu.VMEM((CHUNK + L,), jnp.int32),
            pltpu.VMEM((L,), jnp.int32),
            [pltpu.SemaphoreType.DMA for _ in range(2)],
        ],
    )
    def sc_kernel(ind_hbm, offs_hbm, bounds_hbm, weight_hbm, out_hbm,
                  idx_b, rows_b, staging, cnts, offs_v, bounds_v, delta,
                  bagref, sems):
        wid = lax.axis_index("s") * NC + lax.axis_index("c")
        bag_lo = pl.multiple_of(wid * bags_w, 8)

        zf = jnp.zeros((L,), jnp.float32)
        iota = lax.iota(jnp.int32, L)

        pltpu.sync_copy(offs_hbm.at[pl.ds(bag_lo, bags_w)], offs_v)
        pltpu.sync_copy(bounds_hbm, bounds_v)

        bv = bounds_v[pl.ds(wid, L)]
        row_start = bv[0]
        row_end = bv[1]
        base = lax.bitwise_and(row_start, jnp.int32(-8))
        nchunks = (row_end - base + (CHUNK - 1)) // CHUNK

        def _z(i, _):
            for q in range(dq):
                staging[i, pl.ds(q * L, L)] = zf
            cnts[i, pl.ds(0, L)] = zf
            return 0
        lax.fori_loop(0, stag_rows, _z, 0)

        def issue(g, b):
            r0 = pl.multiple_of(base + g * CHUNK, 8)
            pltpu.sync_copy(ind_hbm.at[pl.ds(r0, CHUNK)], idx_b[b])
            for k in range(GSUB):
                pltpu.async_copy(
                    weight_hbm.at[idx_b[b].at[pl.ds(k * 128, 128)]],
                    rows_b[b].at[pl.ds(k * 128, 128)], sems[b])

        def drain(b):
            for k in range(GSUB):
                pltpu.make_async_copy(
                    weight_hbm.at[idx_b[b].at[pl.ds(k * 128, 128)]],
                    rows_b[b].at[pl.ds(k * 128, 128)], sems[b]).wait()

        def compute(g, b):
            r0 = pl.multiple_of(base + g * CHUNK, 8)
            idx_v = idx_b[b]
            rows_v = rows_b[b]
            bagcum = bagref[pl.ds(0, L)][0]

            def _zd(i, _):
                delta[pl.ds(i * L, L)] = jnp.zeros((L,), jnp.int32)
                return 0
            lax.fori_loop(0, CHUNK // L, _zd, 0)

            for m in range(bags_w // L):
                o = offs_v[pl.ds(m * L, L)]
                inr = jnp.logical_and(o >= r0, o < r0 + CHUNK)
                tgt = jnp.where(inr, o - r0, 0)
                vals = jnp.full((L,), m * L + 1, jnp.int32) + iota
                plsc.store_scatter(delta, [tgt], vals, mask=inr)

            def grp_body(j, gbag):
                rbase = j * L
                d = delta[pl.ds(rbase, L)]
                s = jnp.maximum(plsc.cummax(d), jnp.full((L,), gbag))
                gbag = s[L - 1]
                slot = jnp.where(s == jnp.int32(0), jnp.int32(bags_w), s - 1)
                iv = idx_v[pl.ds(rbase, L)]

                for l in range(L):
                    r = rbase + l
                    vs = [rows_v[r, pl.ds(q * L, L)] for q in range(dq)]
                    sq = [v * v for v in vs]
                    while len(sq) > 1:
                        sq = [sq[i] + sq[i + 1] for i in range(0, len(sq), 2)]
                    n2s = jnp.sum(sq[0]) + 1e-12
                    mv = jnp.where(iv[l] != jnp.int32(0), 1.0, 0.0)
                    mv = mv * jnp.where(r0 + r < row_end, 1.0, 0.0)
                    cv = jnp.minimum(
                        1.0, _rsqrt_newton(jnp.full((L,), n2s))) * mv
                    sl = slot[l]
                    for q in range(dq):
                        plsc.addupdate(staging.at[sl, pl.ds(q * L, L)],
                                       cv * vs[q])
                    plsc.addupdate(cnts.at[sl, pl.ds(0, L)],
                                   jnp.full((L,), mv))
                return gbag

            newbag = lax.fori_loop(0, CHUNK // L, grp_body, bagcum)
            bagref[pl.ds(0, L)] = jnp.full((L,), newbag, jnp.int32)

        bagref[pl.ds(0, L)] = jnp.zeros((L,), jnp.int32)

        @pl.when(nchunks > 0)
        def _():
            issue(0, 0)

        def pair_body(k2, _):
            e = k2 * 2

            @pl.when(e < nchunks)
            def _():
                drain(0)

                @pl.when(e + 1 < nchunks)
                def _():
                    issue(e + 1, 1)
                compute(e, 0)

            @pl.when(e + 1 < nchunks)
            def _():
                drain(1)

                @pl.when(e + 2 < nchunks)
                def _():
                    issue(e + 2, 0)
                compute(e + 1, 1)
            return 0

        lax.fori_loop(0, (nchunks + 1) // 2, pair_body, 0)

        def fin_body(b, _):
            cvec = cnts[b, pl.ds(0, L)]
            ivv = jnp.full((L,), out_scale, jnp.float32) / jnp.maximum(
                cvec, 1.0)
            for q in range(dq):
                staging[b, pl.ds(q * L, L)] = (
                    staging[b, pl.ds(q * L, L)] * ivv)
            return 0
        lax.fori_loop(0, bags_w, fin_body, 0)

        pltpu.sync_copy(staging.at[pl.ds(0, bags_w)],
                        out_hbm.at[pl.ds(bag_lo, bags_w)])

    return sc_kernel


def kernel(indices, offsets, weight):
    n_idx = indices.shape[0]
    n_bags = offsets.shape[0]
    d_model = weight.shape[1]
    bags_w = n_bags // NW

    ind32 = indices.astype(jnp.int32)
    offs32 = offsets.astype(jnp.int32)
    is_last = jnp.concatenate(
        [offs32[1:] != offs32[:-1], jnp.ones((1,), bool)])
    spos = jnp.where(is_last, offs32, -1)
    bounds = jnp.concatenate(
        [offs32[::bags_w], jnp.full((48 - NW,), n_idx, jnp.int32)])
    ind_p = jnp.concatenate([ind32, jnp.zeros((CHUNK,), jnp.int32)])

    sc = _make_sc_kernel(n_idx, n_bags, d_model)
    return sc(ind_p, spos, bounds, weight)
